# baseline (speedup 1.0000x reference)
import functools

import jax
import jax.numpy as jnp
from jax import lax
from jax.experimental import pallas as pl
from jax.experimental.pallas import tpu as pltpu
from jax.experimental.pallas import tpu_sc as plsc

EMB = 64
NLANE = 16
NVEC = EMB // NLANE
NW = 32
GATHER = 128
RU = 4
NB = 4


def _make_sc_kernel(batch: int, seq_len: int):
    chunk = seq_len
    per_w_seq = batch // NW
    per_w = per_w_seq * seq_len
    n_chunk = per_w_seq
    assert batch % NW == 0 and n_chunk % NB == 0 and n_chunk >= 2 * NB
    mesh = plsc.VectorSubcoreMesh(core_axis_name="c", subcore_axis_name="s")

    @functools.partial(
        pl.kernel,
        mesh=mesh,
        compiler_params=pltpu.CompilerParams(needs_layout_passes=False,
                                             use_tc_tiling_on_sc=False),
        out_type=jax.ShapeDtypeStruct((batch, seq_len, EMB), jnp.float32),
        scratch_types=[
            pltpu.VMEM((per_w,), jnp.int32),
            pltpu.VMEM((chunk, EMB), jnp.float32),
            pltpu.VMEM((chunk, EMB), jnp.float32),
            pltpu.VMEM((chunk, EMB), jnp.float32),
            pltpu.VMEM((chunk, EMB), jnp.float32),
            pltpu.VMEM((seq_len, EMB), jnp.float32),
            pltpu.VMEM((EMB,), jnp.float32),
            pltpu.VMEM((EMB,), jnp.float32),
            pltpu.SemaphoreType.DMA,
            pltpu.SemaphoreType.DMA,
            pltpu.SemaphoreType.DMA,
            pltpu.SemaphoreType.DMA,
            pltpu.SemaphoreType.DMA,
            pltpu.SemaphoreType.DMA,
            pltpu.SemaphoreType.DMA,
            pltpu.SemaphoreType.DMA,
        ],
    )
    def sc_kernel(idx_hbm, tok_hbm, pos_hbm, gam_hbm, bet_hbm, out_hbm,
                  idxall_v, rb0, rb1, rb2, rb3, pos_v, gam_v, bet_v,
                  gsem0, gsem1, gsem2, gsem3, osem0, osem1, osem2, osem3):
        wid = lax.axis_index("s") * 2 + lax.axis_index("c")
        base_w = wid * per_w
        seq_w = wid * per_w_seq

        pltpu.sync_copy(idx_hbm.at[pl.ds(base_w, per_w)], idxall_v)
        pltpu.sync_copy(pos_hbm, pos_v)
        pltpu.sync_copy(gam_hbm, gam_v)
        pltpu.sync_copy(bet_hbm, bet_v)
        gv = [gam_v[pl.ds(16 * k, 16)] for k in range(NVEC)]
        bv = [bet_v[pl.ds(16 * k, 16)] for k in range(NVEC)]

        def vrsqrt(v):
            i = lax.bitcast_convert_type(v, jnp.int32)
            i = jnp.int32(0x5F3759DF) - lax.shift_right_arithmetic(i, 1)
            y = lax.bitcast_convert_type(i, jnp.float32)
            vh = 0.5 * v
            for _ in range(2):
                y = y * (1.5 - vh * y * y)
            return y

        bufs = (rb0, rb1, rb2, rb3)
        gsems = (gsem0, gsem1, gsem2, gsem3)
        osems = (osem0, osem1, osem2, osem3)

        def issue_gather(c, b):
            off = c * chunk
            done = 0
            while done < chunk:
                g = min(GATHER, chunk - done)
                pltpu.async_copy(
                    tok_hbm.at[idxall_v.at[pl.ds(off + done, g)]],
                    bufs[b].at[pl.ds(done, g)], gsems[b])
                done += g

        def wait_gather(b):
            pltpu.make_async_copy(tok_hbm.at[pl.ds(0, chunk)],
                                  bufs[b], gsems[b]).wait()

        def issue_out(c, b):
            pltpu.async_copy(bufs[b], out_hbm.at[seq_w + c], osems[b])

        def wait_out(b):
            pltpu.make_async_copy(bufs[b], out_hbm.at[0], osems[b]).wait()

        def compute(b):
            rb = bufs[b]

            @plsc.parallel_loop(0, chunk, step=1, unroll=RU)
            def row_body(r):
                xs = [rb[r, pl.ds(16 * k, 16)] + pos_v[r, pl.ds(16 * k, 16)]
                      for k in range(NVEC)]
                s = (xs[0] + xs[1]) + (xs[2] + xs[3])
                q = (xs[0] * xs[0] + xs[1] * xs[1]) + \
                    (xs[2] * xs[2] + xs[3] * xs[3])
                mean = jnp.sum(s) * (1.0 / EMB)
                var = jnp.sum(q) * (1.0 / EMB) - mean * mean
                rstd = vrsqrt(var + 1e-5)
                m2 = mean * rstd
                for k in range(NVEC):
                    y = xs[k] * rstd - m2
                    rb[r, pl.ds(16 * k, 16)] = y * gv[k] + bv[k]

        for b in range(NB - 1):
            issue_gather(b, b)
        for c in range(NB):
            b = c % NB
            wait_gather(b)
            compute(b)
            issue_out(jnp.int32(c), b)
            bn = (b + NB - 1) % NB
            if c == 0:
                issue_gather(c + NB - 1, bn)
            else:
                wait_out(bn)
                issue_gather(jnp.int32(c + NB - 1), bn)

        def loop_body(i, carry):
            c0 = NB * i
            for b in range(NB):
                c = c0 + b
                wait_gather(b)
                compute(b)
                issue_out(c, b)
                bn = (b + NB - 1) % NB
                wait_out(bn)
                issue_gather(jnp.minimum(c + NB - 1, n_chunk - 1), bn)
            return carry

        lax.fori_loop(1, n_chunk // NB, loop_body, 0)

        for k in range(NB - 1):
            wait_gather(k % NB)
        wait_out((n_chunk - 1) % NB)

    return sc_kernel


def kernel(input_tensor, res_mask, token_table, position_table, gamma, beta):
    batch, seq_len = input_tensor.shape
    idx_flat = input_tensor.reshape(batch * seq_len).astype(jnp.int32)
    pos_used = position_table[:seq_len]
    return _make_sc_kernel(batch, seq_len)(
        idx_flat, token_table, pos_used, gamma, beta)


# speedup vs baseline: 1.0036x; 1.0036x over previous
import functools

import jax
import jax.numpy as jnp
from jax import lax
from jax.experimental import pallas as pl
from jax.experimental.pallas import tpu as pltpu
from jax.experimental.pallas import tpu_sc as plsc

EMB = 64
NLANE = 16
NVEC = EMB // NLANE
NW = 32
GATHER = 128
RU = 8


def _make_sc_kernel(batch: int, seq_len: int):
    chunk = seq_len
    per_w_seq = batch // NW
    per_w = per_w_seq * seq_len
    n_chunk = per_w_seq
    assert batch % NW == 0 and n_chunk % 2 == 0 and seq_len % 8 == 0
    mesh = plsc.VectorSubcoreMesh(core_axis_name="c", subcore_axis_name="s")

    @functools.partial(
        pl.kernel,
        mesh=mesh,
        compiler_params=pltpu.CompilerParams(needs_layout_passes=False,
                                             use_tc_tiling_on_sc=False),
        out_type=jax.ShapeDtypeStruct((batch, seq_len, EMB), jnp.float32),
        scratch_types=[
            pltpu.VMEM((per_w,), jnp.int32),
            pltpu.VMEM((chunk, EMB), jnp.float32),
            pltpu.VMEM((chunk, EMB), jnp.float32),
            pltpu.VMEM((chunk, EMB), jnp.float32),
            pltpu.VMEM((chunk, EMB), jnp.float32),
            pltpu.VMEM((seq_len, EMB), jnp.float32),
            pltpu.VMEM((EMB,), jnp.float32),
            pltpu.VMEM((EMB,), jnp.float32),
            pltpu.SemaphoreType.DMA,
            pltpu.SemaphoreType.DMA,
            pltpu.SemaphoreType.DMA,
            pltpu.SemaphoreType.DMA,
        ],
    )
    def sc_kernel(idx_hbm, tok_hbm, pos_hbm, gam_hbm, bet_hbm, out_hbm,
                  idxall_v, ib0, ib1, ob0, ob1, pos_v, gam_v, bet_v,
                  gsem0, gsem1, osem0, osem1):
        wid = lax.axis_index("s") * 2 + lax.axis_index("c")
        base_w = wid * per_w
        seq_w = wid * per_w_seq

        pltpu.sync_copy(idx_hbm.at[pl.ds(base_w, per_w)], idxall_v)
        pltpu.sync_copy(pos_hbm, pos_v)
        pltpu.sync_copy(gam_hbm, gam_v)
        pltpu.sync_copy(bet_hbm, bet_v)
        gv = [gam_v[pl.ds(16 * k, 16)] for k in range(NVEC)]
        bv = [bet_v[pl.ds(16 * k, 16)] for k in range(NVEC)]

        def vrsqrt(v):
            i = lax.bitcast_convert_type(v, jnp.int32)
            i = jnp.int32(0x5F3759DF) - lax.shift_right_arithmetic(i, 1)
            y = lax.bitcast_convert_type(i, jnp.float32)
            vh = 0.5 * v
            for _ in range(2):
                y = y * (1.5 - vh * y * y)
            return y

        ibufs, obufs = (ib0, ib1), (ob0, ob1)
        gsems, osems = (gsem0, gsem1), (osem0, osem1)

        def issue_gather(c, b):
            off = c * chunk
            done = 0
            while done < chunk:
                g = min(GATHER, chunk - done)
                pltpu.async_copy(
                    tok_hbm.at[idxall_v.at[pl.ds(off + done, g)]],
                    ibufs[b].at[pl.ds(done, g)], gsems[b])
                done += g

        def wait_gather(b):
            pltpu.make_async_copy(tok_hbm.at[pl.ds(0, chunk)],
                                  ibufs[b], gsems[b]).wait()

        def issue_out(c, b):
            pltpu.async_copy(obufs[b], out_hbm.at[seq_w + c], osems[b])

        def wait_out(b):
            pltpu.make_async_copy(obufs[b], out_hbm.at[0], osems[b]).wait()

        def compute(b):
            ib, ob = ibufs[b], obufs[b]

            @plsc.parallel_loop(0, chunk, step=1, unroll=RU)
            def row_body(r):
                xs = [ib[r, pl.ds(16 * k, 16)] + pos_v[r, pl.ds(16 * k, 16)]
                      for k in range(NVEC)]
                s = (xs[0] + xs[1]) + (xs[2] + xs[3])
                q = (xs[0] * xs[0] + xs[1] * xs[1]) + \
                    (xs[2] * xs[2] + xs[3] * xs[3])
                mean = jnp.sum(s) * (1.0 / EMB)
                var = jnp.sum(q) * (1.0 / EMB) - mean * mean
                rstd = vrsqrt(var + 1e-5)
                m2 = mean * rstd
                for k in range(NVEC):
                    y = xs[k] * rstd - m2
                    ob[r, pl.ds(16 * k, 16)] = y * gv[k] + bv[k]

        issue_gather(0, 0)
        issue_gather(1, 1)
        for b in (0, 1):
            wait_gather(b)
            compute(b)
            issue_out(jnp.int32(b), b)
            issue_gather(jnp.int32(b + 2), b)

        def loop_body(i, carry):
            c0 = 2 * i
            for b in (0, 1):
                c = c0 + b
                wait_gather(b)
                wait_out(b)
                compute(b)
                issue_out(c, b)
                issue_gather(jnp.minimum(c + 2, n_chunk - 1), b)
            return carry

        lax.fori_loop(1, n_chunk // 2, loop_body, 0)

        for b in (0, 1):
            wait_gather(b)
            wait_out(b)

    return sc_kernel


def kernel(input_tensor, res_mask, token_table, position_table, gamma, beta):
    batch, seq_len = input_tensor.shape
    idx_flat = input_tensor.reshape(batch * seq_len).astype(jnp.int32)
    pos_used = position_table[:seq_len]
    return _make_sc_kernel(batch, seq_len)(
        idx_flat, token_table, pos_used, gamma, beta)
